# baseline (speedup 1.0000x reference)
import functools

import jax
import jax.numpy as jnp
from jax import lax
from jax.experimental import pallas as pl
from jax.experimental.pallas import tpu as pltpu
from jax.experimental.pallas import tpu_sc as plsc

TEMP = 1.0
THRESHOLD = 0.5
PRIOR = 0.2
EPS = 1e-8

B, S, D, H = 4, 2048, 1024, 2048
SBLK = 256
NSB = S // SBLK
GP = 2056
NTILES = 32
CHUNK = S // (NTILES // B)
SUB = 16

_HI = jax.lax.Precision.HIGHEST


def _stage1_body(x_ref, w1_ref, b1_ref, w2_ref, logits_ref, c_ref, carry_ref):
    x = x_ref[0]
    xb = x.astype(jnp.bfloat16)
    h = jax.lax.dot_general(xb, w1_ref[...], (((1,), (0,)), ((), ())),
                            preferred_element_type=jnp.float32)
    h = jnp.maximum(h + b1_ref[...], 0.0)
    hb = h.astype(jnp.bfloat16).astype(jnp.float32)
    w2f = w2_ref[...].astype(jnp.float32)
    logits = jnp.sum(hb * w2f, axis=1)
    logits_ref[...] = logits.reshape(1, 1, 1, SBLK)

    r = jax.lax.broadcasted_iota(jnp.int32, (SBLK, SBLK), 0)
    c = jax.lax.broadcasted_iota(jnp.int32, (SBLK, SBLK), 1)
    ltri = (r >= c).astype(jnp.bfloat16)
    cs = jax.lax.dot_general(ltri, xb, (((1,), (0,)), ((), ())),
                             preferred_element_type=jnp.float32)

    @pl.when(pl.program_id(1) == 0)
    def _():
        carry_ref[...] = jnp.zeros_like(carry_ref)

    total = cs + carry_ref[...]
    c_ref[0] = total
    carry_ref[...] = total[SBLK - 1:SBLK, :]


def _stage1(hidden, W1b, b1r, w2rb):
    return pl.pallas_call(
        _stage1_body,
        grid=(B, NSB),
        in_specs=[
            pl.BlockSpec((1, SBLK, D), lambda b, s: (b, s, 0)),
            pl.BlockSpec((D, H), lambda b, s: (0, 0)),
            pl.BlockSpec((1, H), lambda b, s: (0, 0)),
            pl.BlockSpec((1, H), lambda b, s: (0, 0)),
        ],
        out_specs=[
            pl.BlockSpec((1, 1, 1, SBLK), lambda b, s: (b, s, 0, 0)),
            pl.BlockSpec((1, SBLK, D), lambda b, s: (b, s, 0)),
        ],
        out_shape=[
            jax.ShapeDtypeStruct((B, NSB, 1, SBLK), jnp.float32),
            jax.ShapeDtypeStruct((B, S, D), jnp.float32),
        ],
        scratch_shapes=[pltpu.VMEM((1, D), jnp.float32)],
        compiler_params=pltpu.CompilerParams(
            dimension_semantics=("arbitrary", "arbitrary")),
    )(hidden, W1b, b1r, w2rb)


def _lane_cumsum(x):
    n = x.shape[1]
    sh = 1
    while sh < n:
        x = x + jnp.concatenate(
            [jnp.zeros(x.shape[:1] + (sh,), x.dtype), x[:, :-sh]], axis=1)
        sh *= 2
    return x


def _stage2_body(logits_ref, u_ref, b2_ref, g_ref, inv_ref, nb_ref):
    logits = logits_ref[...] + b2_ref[0, 0]
    probs = 1.0 / (1.0 + jnp.exp(-logits))
    u = u_ref[...]
    noisy = (jnp.log(probs + EPS) - jnp.log(1.0 - probs + EPS)
             + jnp.log(u) - jnp.log(1.0 - u)) / TEMP
    soft = 1.0 / (1.0 + jnp.exp(-noisy))
    hard = (soft > THRESHOLD)
    hardi = hard.astype(jnp.int32)
    nb_ref[0, 0] = jnp.sum(hardi.astype(jnp.float32))

    seg = _lane_cumsum(hardi) - hardi

    for b in range(B):
        segb = seg[b:b + 1, :]
        e_cols = []
        for k in range(NSB):
            nv = (jax.lax.broadcasted_iota(jnp.int32, (SBLK, 1), 0)
                  + k * SBLK)
            cmp = (segb <= nv).astype(jnp.int32)
            e_cols.append(jnp.sum(cmp, axis=1, keepdims=True))
        e_b = jnp.concatenate(e_cols, axis=0)
        base = b * (S + 1)
        g_ref[b] = base + e_b
        e_prev = jnp.concatenate(
            [jnp.zeros((1, 1), jnp.int32), e_b[:-1]], axis=0)
        cnt = (e_b - e_prev).astype(jnp.float32)
        inv_ref[b] = jnp.broadcast_to(1.0 / (cnt + 1e-9), (S, 16))


def _stage2(logits, u, b2r):
    return pl.pallas_call(
        _stage2_body,
        in_specs=[
            pl.BlockSpec((B, S), lambda: (0, 0)),
            pl.BlockSpec((B, S), lambda: (0, 0)),
            pl.BlockSpec(memory_space=pltpu.SMEM),
        ],
        out_specs=[
            pl.BlockSpec((B, S, 1), lambda: (0, 0, 0)),
            pl.BlockSpec((B, S, 16), lambda: (0, 0, 0)),
            pl.BlockSpec(memory_space=pltpu.SMEM),
        ],
        out_shape=[
            jax.ShapeDtypeStruct((B, S, 1), jnp.int32),
            jax.ShapeDtypeStruct((B, S, 16), jnp.float32),
            jax.ShapeDtypeStruct((1, 1), jnp.float32),
        ],
    )(logits, u, b2r)


def _sc_body(cpad_hbm, g_hbm, gs_hbm, inv_hbm, out_hbm, idx_v, sidx_v, inv_v,
             rows_a, rows_b, seed_v, outb_v, sem_a, sem_b):
    wid = lax.axis_index("s") * 2 + lax.axis_index("c")
    nk = CHUNK // SUB
    ibase = wid * CHUNK
    pltpu.sync_copy(g_hbm.at[pl.ds(wid * nk, nk)], idx_v)
    pltpu.sync_copy(gs_hbm.at[pl.ds(wid, 1)], sidx_v)
    pltpu.sync_copy(inv_hbm.at[pl.ds(ibase, CHUNK)], inv_v)
    pltpu.async_copy(cpad_hbm.at[sidx_v.at[0]], seed_v, sem_a).wait()
    bufs = (rows_a, rows_b)
    sems = (sem_a, sem_b)
    h = pltpu.async_copy(cpad_hbm.at[idx_v.at[0]], rows_a, sem_a)
    for k in range(nk):
        cur = bufs[k % 2]
        prv = bufs[(k + 1) % 2]
        h.wait()
        inv0 = inv_v[k * SUB, :]

        def first_body(c, carry, _cur=cur, _prv=prv, _k=k, _inv0=inv0):
            off = pl.multiple_of(c * 16, 16)
            lo = (seed_v[7, pl.ds(off, 16)] if _k == 0
                  else _prv[SUB - 1, pl.ds(off, 16)])
            outb_v[0, pl.ds(off, 16)] = (_cur[0, pl.ds(off, 16)] - lo) * _inv0
            return carry

        lax.fori_loop(0, D // 16, first_body, 0)

        if k + 1 < nk:
            h = pltpu.async_copy(cpad_hbm.at[idx_v.at[k + 1]],
                                 bufs[(k + 1) % 2], sems[(k + 1) % 2])

        def row_body(i, carry, _cur=cur, _k=k):
            invb = inv_v[_k * SUB + i, :]

            def lane_body(c, carry2):
                off = pl.multiple_of(c * 16, 16)
                lo = _cur[i - 1, pl.ds(off, 16)]
                hi = _cur[i, pl.ds(off, 16)]
                outb_v[i, pl.ds(off, 16)] = (hi - lo) * invb
                return carry2

            return lax.fori_loop(0, D // 16, lane_body, carry)

        lax.fori_loop(1, SUB, row_body, 0)
        pltpu.sync_copy(outb_v, out_hbm.at[pl.ds(ibase + k * SUB, SUB)])


def _sc_gather(cpad_flat, g_win, g_seed, inv_bc):
    mesh = plsc.VectorSubcoreMesh(core_axis_name="c", subcore_axis_name="s")
    f = functools.partial(
        pl.kernel,
        mesh=mesh,
        out_type=jax.ShapeDtypeStruct((B * S, D), jnp.float32),
        scratch_types=[
            pltpu.VMEM((CHUNK // SUB, SUB), jnp.int32),
            pltpu.VMEM((1, 8), jnp.int32),
            pltpu.VMEM((CHUNK, 16), jnp.float32),
            pltpu.VMEM((SUB, D), jnp.float32),
            pltpu.VMEM((SUB, D), jnp.float32),
            pltpu.VMEM((8, D), jnp.float32),
            pltpu.VMEM((SUB, D), jnp.float32),
            pltpu.SemaphoreType.DMA,
            pltpu.SemaphoreType.DMA,
        ],
    )(_sc_body)
    return f(cpad_flat, g_win, g_seed, inv_bc)


def kernel(hidden, W1, b1, W2, b2):
    u = jax.random.uniform(jax.random.key(42), (B, S),
                           minval=1e-6, maxval=1.0 - 1e-6, dtype=jnp.float32)
    logits4, C = _stage1(hidden, W1.astype(jnp.bfloat16), b1.reshape(1, H),
                         W2.reshape(1, H).astype(jnp.bfloat16))
    logits = logits4.reshape(B, S)
    G2, invbc, nb = _stage2(logits, u, b2.reshape(1, 1))

    bases = (jnp.arange(B, dtype=jnp.int32) * (S + 1))[:, None]
    gpad = jnp.concatenate([bases, G2.reshape(B, S)], axis=1)
    gwin = gpad[:, 1:].reshape(B * (S // SUB), SUB)
    gseed = jnp.broadcast_to(
        gpad[:, 0:S:CHUNK].reshape(NTILES, 1), (NTILES, 8))

    cpad = jnp.concatenate(
        [jnp.zeros((B, 1, D), jnp.float32), C], axis=1).reshape(B * (S + 1), D)
    pooled = _sc_gather(cpad, gwin, gseed,
                        invbc.reshape(B * S, 16)).reshape(B, S, D)

    k = nb[0, 0]
    n = jnp.asarray(B * S, jnp.float32)
    log_prob = (jax.lax.lgamma(n + 1.0) - jax.lax.lgamma(k + 1.0)
                - jax.lax.lgamma(n - k + 1.0)
                + k * jnp.log(PRIOR) + (n - k) * jnp.log(1.0 - PRIOR))
    loss = -log_prob / n
    return pooled, loss


# speedup vs baseline: 1.1727x; 1.1727x over previous
import functools

import jax
import jax.numpy as jnp
from jax import lax
from jax.experimental import pallas as pl
from jax.experimental.pallas import tpu as pltpu
from jax.experimental.pallas import tpu_sc as plsc

TEMP = 1.0
THRESHOLD = 0.5
PRIOR = 0.2
EPS = 1e-8

B, S, D, H = 4, 2048, 1024, 2048
SBLK = 256
NSB = S // SBLK
GP = 2056
NTILES = 32
CHUNK = S // (NTILES // B)
SUB = 32

_HI = jax.lax.Precision.HIGHEST


def _stage1_body(x_ref, w1_ref, b1_ref, w2_ref, logits_ref, c_ref, carry_ref):
    x = x_ref[0]
    xb = x.astype(jnp.bfloat16)
    h = jax.lax.dot_general(xb, w1_ref[...], (((1,), (0,)), ((), ())),
                            preferred_element_type=jnp.float32)
    h = jnp.maximum(h + b1_ref[...], 0.0)
    hb = h.astype(jnp.bfloat16).astype(jnp.float32)
    w2f = w2_ref[...].astype(jnp.float32)
    logits = jnp.sum(hb * w2f, axis=1)
    logits_ref[...] = logits.reshape(1, 1, 1, SBLK)

    r = jax.lax.broadcasted_iota(jnp.int32, (SBLK, SBLK), 0)
    c = jax.lax.broadcasted_iota(jnp.int32, (SBLK, SBLK), 1)
    ltri = (r >= c).astype(jnp.bfloat16)
    cs = jax.lax.dot_general(ltri, xb, (((1,), (0,)), ((), ())),
                             preferred_element_type=jnp.float32)

    @pl.when(pl.program_id(1) == 0)
    def _():
        carry_ref[...] = jnp.zeros_like(carry_ref)

    total = cs + carry_ref[...]
    c_ref[0] = total
    carry_ref[...] = total[SBLK - 1:SBLK, :]


def _stage1(hidden, W1b, b1r, w2rb):
    return pl.pallas_call(
        _stage1_body,
        grid=(B, NSB),
        in_specs=[
            pl.BlockSpec((1, SBLK, D), lambda b, s: (b, s, 0)),
            pl.BlockSpec((D, H), lambda b, s: (0, 0)),
            pl.BlockSpec((1, H), lambda b, s: (0, 0)),
            pl.BlockSpec((1, H), lambda b, s: (0, 0)),
        ],
        out_specs=[
            pl.BlockSpec((1, 1, 1, SBLK), lambda b, s: (b, s, 0, 0)),
            pl.BlockSpec((1, SBLK, D), lambda b, s: (b, s, 0)),
        ],
        out_shape=[
            jax.ShapeDtypeStruct((B, NSB, 1, SBLK), jnp.float32),
            jax.ShapeDtypeStruct((B, S, D), jnp.float32),
        ],
        scratch_shapes=[pltpu.VMEM((1, D), jnp.float32)],
        compiler_params=pltpu.CompilerParams(
            dimension_semantics=("arbitrary", "arbitrary")),
    )(hidden, W1b, b1r, w2rb)


def _lane_cumsum(x):
    n = x.shape[1]
    sh = 1
    while sh < n:
        x = x + jnp.concatenate(
            [jnp.zeros(x.shape[:1] + (sh,), x.dtype), x[:, :-sh]], axis=1)
        sh *= 2
    return x


def _stage2_body(logits_ref, u_ref, b2_ref, g_ref, inv_ref, nb_ref):
    logits = logits_ref[...] + b2_ref[0, 0]
    probs = 1.0 / (1.0 + jnp.exp(-logits))
    u = u_ref[...]
    noisy = (jnp.log(probs + EPS) - jnp.log(1.0 - probs + EPS)
             + jnp.log(u) - jnp.log(1.0 - u)) / TEMP
    soft = 1.0 / (1.0 + jnp.exp(-noisy))
    hard = (soft > THRESHOLD)
    hardi = hard.astype(jnp.int32)
    nb_ref[0, 0] = jnp.sum(hardi.astype(jnp.float32))

    seg = _lane_cumsum(hardi) - hardi

    for b in range(B):
        segb = seg[b:b + 1, :]
        e_cols = []
        for k in range(NSB):
            nv = (jax.lax.broadcasted_iota(jnp.int32, (SBLK, 1), 0)
                  + k * SBLK)
            cmp = (segb <= nv).astype(jnp.int32)
            e_cols.append(jnp.sum(cmp, axis=1, keepdims=True))
        e_b = jnp.concatenate(e_cols, axis=0)
        base = b * (S + 1)
        g_ref[b] = base + e_b
        e_prev = jnp.concatenate(
            [jnp.zeros((1, 1), jnp.int32), e_b[:-1]], axis=0)
        cnt = (e_b - e_prev).astype(jnp.float32)
        inv_ref[b] = 1.0 / (cnt + 1e-9)


def _stage2(logits, u, b2r):
    return pl.pallas_call(
        _stage2_body,
        in_specs=[
            pl.BlockSpec((B, S), lambda: (0, 0)),
            pl.BlockSpec((B, S), lambda: (0, 0)),
            pl.BlockSpec(memory_space=pltpu.SMEM),
        ],
        out_specs=[
            pl.BlockSpec((B, S, 1), lambda: (0, 0, 0)),
            pl.BlockSpec((B, S, 1), lambda: (0, 0, 0)),
            pl.BlockSpec(memory_space=pltpu.SMEM),
        ],
        out_shape=[
            jax.ShapeDtypeStruct((B, S, 1), jnp.int32),
            jax.ShapeDtypeStruct((B, S, 1), jnp.float32),
            jax.ShapeDtypeStruct((1, 1), jnp.float32),
        ],
    )(logits, u, b2r)


def _sc_body(cpad_hbm, g_hbm, out_hbm, idx_v, rows_a, rows_b,
             sem_a, sem_b, semw_a, semw_b):
    wid = lax.axis_index("s") * 2 + lax.axis_index("c")
    nk = CHUNK // SUB
    ibase = wid * CHUNK
    pltpu.sync_copy(g_hbm.at[pl.ds(wid * nk, nk)], idx_v)
    bufs = (rows_a, rows_b)
    sems = (sem_a, sem_b)
    semw = (semw_a, semw_b)
    wpend = [None, None]
    h = pltpu.async_copy(cpad_hbm.at[idx_v.at[0]], rows_a, sem_a)
    for k in range(nk):
        cur = bufs[k % 2]
        h.wait()
        if k + 1 < nk:
            nxt = (k + 1) % 2
            if wpend[nxt] is not None:
                wpend[nxt].wait()
                wpend[nxt] = None
            h = pltpu.async_copy(cpad_hbm.at[idx_v.at[k + 1]],
                                 bufs[nxt], sems[nxt])
        wpend[k % 2] = pltpu.async_copy(
            cur, out_hbm.at[pl.ds(ibase + k * SUB, SUB)], semw[k % 2])
    for w in wpend:
        if w is not None:
            w.wait()


def _sc_gather(cpad_flat, g_win):
    mesh = plsc.VectorSubcoreMesh(core_axis_name="c", subcore_axis_name="s")
    f = functools.partial(
        pl.kernel,
        mesh=mesh,
        out_type=jax.ShapeDtypeStruct((B * S, D), jnp.float32),
        scratch_types=[
            pltpu.VMEM((CHUNK // SUB, SUB), jnp.int32),
            pltpu.VMEM((SUB, D), jnp.float32),
            pltpu.VMEM((SUB, D), jnp.float32),
            pltpu.SemaphoreType.DMA,
            pltpu.SemaphoreType.DMA,
            pltpu.SemaphoreType.DMA,
            pltpu.SemaphoreType.DMA,
        ],
    )(_sc_body)
    return f(cpad_flat, g_win)


def _stage4_body(g_ref, seed_ref, inv_ref, out_ref):
    x = g_ref[0]
    prev = seed_ref[0, 7:8, :]
    prev = jnp.where(pl.program_id(1) == 0, 0.0, prev)
    shifted = jnp.concatenate([prev, x[:SBLK - 1, :]], axis=0)
    out_ref[0] = (x - shifted) * inv_ref[0]


def _stage4(gath, inv_col):
    return pl.pallas_call(
        _stage4_body,
        grid=(B, NSB),
        in_specs=[
            pl.BlockSpec((1, SBLK, D), lambda b, s: (b, s, 0)),
            pl.BlockSpec((1, 8, D),
                         lambda b, s: (b, jnp.maximum(s * (SBLK // 8) - 1, 0),
                                       0)),
            pl.BlockSpec((1, SBLK, 1), lambda b, s: (b, s, 0)),
        ],
        out_specs=pl.BlockSpec((1, SBLK, D), lambda b, s: (b, s, 0)),
        out_shape=jax.ShapeDtypeStruct((B, S, D), jnp.float32),
    )(gath, gath, inv_col)


def kernel(hidden, W1, b1, W2, b2):
    u = jax.random.uniform(jax.random.key(42), (B, S),
                           minval=1e-6, maxval=1.0 - 1e-6, dtype=jnp.float32)
    logits4, C = _stage1(hidden, W1.astype(jnp.bfloat16), b1.reshape(1, H),
                         W2.reshape(1, H).astype(jnp.bfloat16))
    logits = logits4.reshape(B, S)
    G2, invbc, nb = _stage2(logits, u, b2.reshape(1, 1))

    gwin = G2.reshape(B * (S // SUB), SUB)

    cpad = jnp.concatenate(
        [jnp.zeros((B, 1, D), jnp.float32), C], axis=1).reshape(B * (S + 1), D)
    gath = _sc_gather(cpad, gwin).reshape(B, S, D)
    pooled = _stage4(gath, invbc)

    k = nb[0, 0]
    n = jnp.asarray(B * S, jnp.float32)
    log_prob = (jax.lax.lgamma(n + 1.0) - jax.lax.lgamma(k + 1.0)
                - jax.lax.lgamma(n - k + 1.0)
                + k * jnp.log(PRIOR) + (n - k) * jnp.log(1.0 - PRIOR))
    loss = -log_prob / n
    return pooled, loss


# speedup vs baseline: 1.4778x; 1.2602x over previous
import functools

import jax
import jax.numpy as jnp
from jax import lax
from jax.experimental import pallas as pl
from jax.experimental.pallas import tpu as pltpu
from jax.experimental.pallas import tpu_sc as plsc

TEMP = 1.0
THRESHOLD = 0.5
PRIOR = 0.2
EPS = 1e-8

B, S, D, H = 4, 2048, 1024, 2048
SBLK = 256
NSB = S // SBLK
GP = 2056
NTILES = 32
CHUNK = S // (NTILES // B)
SUB = 32

_HI = jax.lax.Precision.HIGHEST


def _stage1_body(x_ref, w1_ref, b1_ref, w2_ref, logits_ref, c_ref, carry_ref):
    x = x_ref[0]
    xb = x.astype(jnp.bfloat16)
    h = jax.lax.dot_general(xb, w1_ref[...], (((1,), (0,)), ((), ())),
                            preferred_element_type=jnp.float32)
    h = jnp.maximum(h + b1_ref[...], 0.0)
    hb = h.astype(jnp.bfloat16).astype(jnp.float32)
    w2f = w2_ref[...].astype(jnp.float32)
    logits = jnp.sum(hb * w2f, axis=1)
    logits_ref[...] = logits.reshape(1, 1, 1, SBLK)

    r = jax.lax.broadcasted_iota(jnp.int32, (SBLK, SBLK), 0)
    c = jax.lax.broadcasted_iota(jnp.int32, (SBLK, SBLK), 1)
    ltri = (r >= c).astype(jnp.bfloat16)
    cs = jax.lax.dot_general(ltri, xb, (((1,), (0,)), ((), ())),
                             preferred_element_type=jnp.float32)

    @pl.when(pl.program_id(1) == 0)
    def _():
        carry_ref[...] = jnp.zeros_like(carry_ref)

    total = cs + carry_ref[...]
    c_ref[0] = total
    carry_ref[...] = total[SBLK - 1:SBLK, :]


def _stage1(hidden, W1b, b1r, w2rb):
    return pl.pallas_call(
        _stage1_body,
        grid=(B, NSB),
        in_specs=[
            pl.BlockSpec((1, SBLK, D), lambda b, s: (b, s, 0)),
            pl.BlockSpec((D, H), lambda b, s: (0, 0)),
            pl.BlockSpec((1, H), lambda b, s: (0, 0)),
            pl.BlockSpec((1, H), lambda b, s: (0, 0)),
        ],
        out_specs=[
            pl.BlockSpec((1, 1, 1, SBLK), lambda b, s: (b, s, 0, 0)),
            pl.BlockSpec((1, SBLK, D), lambda b, s: (b, s, 0)),
        ],
        out_shape=[
            jax.ShapeDtypeStruct((B, NSB, 1, SBLK), jnp.float32),
            jax.ShapeDtypeStruct((B, S, D), jnp.float32),
        ],
        scratch_shapes=[pltpu.VMEM((1, D), jnp.float32)],
        compiler_params=pltpu.CompilerParams(
            dimension_semantics=("arbitrary", "arbitrary")),
    )(hidden, W1b, b1r, w2rb)


def _lane_cumsum(x):
    n = x.shape[1]
    sh = 1
    while sh < n:
        x = x + jnp.concatenate(
            [jnp.zeros(x.shape[:1] + (sh,), x.dtype), x[:, :-sh]], axis=1)
        sh *= 2
    return x


def _stage2_body(logits_ref, u_ref, b2_ref, g_ref, inv_ref, nb_ref):
    logits = logits_ref[...] + b2_ref[0, 0]
    probs = 1.0 / (1.0 + jnp.exp(-logits))
    u = u_ref[...]
    noisy = (jnp.log(probs + EPS) - jnp.log(1.0 - probs + EPS)
             + jnp.log(u) - jnp.log(1.0 - u)) / TEMP
    soft = 1.0 / (1.0 + jnp.exp(-noisy))
    hard = (soft > THRESHOLD)
    hardi = hard.astype(jnp.int32)
    nb_ref[0, 0] = jnp.sum(hardi.astype(jnp.float32))

    seg = _lane_cumsum(hardi) - hardi

    for b in range(B):
        segb = seg[b:b + 1, :]
        e_cols = []
        for k in range(NSB):
            nv = (jax.lax.broadcasted_iota(jnp.int32, (SBLK, 1), 0)
                  + k * SBLK)
            cmp = (segb <= nv).astype(jnp.int32)
            e_cols.append(jnp.sum(cmp, axis=1, keepdims=True))
        e_b = jnp.concatenate(e_cols, axis=0)
        g_ref[b] = (b * S - 1) + e_b
        e_prev = jnp.concatenate(
            [jnp.zeros((1, 1), jnp.int32), e_b[:-1]], axis=0)
        cnt = (e_b - e_prev).astype(jnp.float32)
        inv_ref[b] = 1.0 / (cnt + 1e-9)


def _stage2(logits, u, b2r):
    return pl.pallas_call(
        _stage2_body,
        in_specs=[
            pl.BlockSpec((B, S), lambda: (0, 0)),
            pl.BlockSpec((B, S), lambda: (0, 0)),
            pl.BlockSpec(memory_space=pltpu.SMEM),
        ],
        out_specs=[
            pl.BlockSpec((B, S, 1), lambda: (0, 0, 0)),
            pl.BlockSpec((B, S, 1), lambda: (0, 0, 0)),
            pl.BlockSpec(memory_space=pltpu.SMEM),
        ],
        out_shape=[
            jax.ShapeDtypeStruct((B, S, 1), jnp.int32),
            jax.ShapeDtypeStruct((B, S, 1), jnp.float32),
            jax.ShapeDtypeStruct((1, 1), jnp.float32),
        ],
    )(logits, u, b2r)


def _sc_body(cpad_hbm, g_hbm, out_hbm, idx_v, rows_a, rows_b,
             sem_a, sem_b, semw_a, semw_b):
    wid = lax.axis_index("s") * 2 + lax.axis_index("c")
    nk = CHUNK // SUB
    ibase = wid * CHUNK
    pltpu.sync_copy(g_hbm.at[pl.ds(wid * nk, nk)], idx_v)
    bufs = (rows_a, rows_b)
    sems = (sem_a, sem_b)
    semw = (semw_a, semw_b)
    wpend = [None, None]
    h = pltpu.async_copy(cpad_hbm.at[idx_v.at[0]], rows_a, sem_a)
    for k in range(nk):
        cur = bufs[k % 2]
        h.wait()
        if k + 1 < nk:
            nxt = (k + 1) % 2
            if wpend[nxt] is not None:
                wpend[nxt].wait()
                wpend[nxt] = None
            h = pltpu.async_copy(cpad_hbm.at[idx_v.at[k + 1]],
                                 bufs[nxt], sems[nxt])
        wpend[k % 2] = pltpu.async_copy(
            cur, out_hbm.at[pl.ds(ibase + k * SUB, SUB)], semw[k % 2])
    for w in wpend:
        if w is not None:
            w.wait()


def _sc_gather(cpad_flat, g_win):
    mesh = plsc.VectorSubcoreMesh(core_axis_name="c", subcore_axis_name="s")
    f = functools.partial(
        pl.kernel,
        mesh=mesh,
        out_type=jax.ShapeDtypeStruct((B * S, D), jnp.float32),
        scratch_types=[
            pltpu.VMEM((CHUNK // SUB, SUB), jnp.int32),
            pltpu.VMEM((SUB, D), jnp.float32),
            pltpu.VMEM((SUB, D), jnp.float32),
            pltpu.SemaphoreType.DMA,
            pltpu.SemaphoreType.DMA,
            pltpu.SemaphoreType.DMA,
            pltpu.SemaphoreType.DMA,
        ],
    )(_sc_body)
    return f(cpad_flat, g_win)


def _stage4_body(g_ref, seed_ref, inv_ref, out_ref):
    x = g_ref[0]
    prev = seed_ref[0, 7:8, :]
    prev = jnp.where(pl.program_id(1) == 0, 0.0, prev)
    shifted = jnp.concatenate([prev, x[:SBLK - 1, :]], axis=0)
    out_ref[0] = (x - shifted) * inv_ref[0]


def _stage4(gath, inv_col):
    return pl.pallas_call(
        _stage4_body,
        grid=(B, NSB),
        in_specs=[
            pl.BlockSpec((1, SBLK, D), lambda b, s: (b, s, 0)),
            pl.BlockSpec((1, 8, D),
                         lambda b, s: (b, jnp.maximum(s * (SBLK // 8) - 1, 0),
                                       0)),
            pl.BlockSpec((1, SBLK, 1), lambda b, s: (b, s, 0)),
        ],
        out_specs=pl.BlockSpec((1, SBLK, D), lambda b, s: (b, s, 0)),
        out_shape=jax.ShapeDtypeStruct((B, S, D), jnp.float32),
    )(gath, gath, inv_col)


def kernel(hidden, W1, b1, W2, b2):
    u = jax.random.uniform(jax.random.key(42), (B, S),
                           minval=1e-6, maxval=1.0 - 1e-6, dtype=jnp.float32)
    logits4, C = _stage1(hidden, W1.astype(jnp.bfloat16), b1.reshape(1, H),
                         W2.reshape(1, H).astype(jnp.bfloat16))
    logits = logits4.reshape(B, S)
    G2, invbc, nb = _stage2(logits, u, b2.reshape(1, 1))

    gwin = G2.reshape(B * (S // SUB), SUB)
    gath = _sc_gather(C.reshape(B * S, D), gwin).reshape(B, S, D)
    pooled = _stage4(gath, invbc)

    k = nb[0, 0]
    n = jnp.asarray(B * S, jnp.float32)
    log_prob = (jax.lax.lgamma(n + 1.0) - jax.lax.lgamma(k + 1.0)
                - jax.lax.lgamma(n - k + 1.0)
                + k * jnp.log(PRIOR) + (n - k) * jnp.log(1.0 - PRIOR))
    loss = -log_prob / n
    return pooled, loss


# speedup vs baseline: 1.5473x; 1.0470x over previous
import functools

import jax
import jax.numpy as jnp
from jax import lax
from jax.experimental import pallas as pl
from jax.experimental.pallas import tpu as pltpu
from jax.experimental.pallas import tpu_sc as plsc

TEMP = 1.0
THRESHOLD = 0.5
PRIOR = 0.2
EPS = 1e-8

B, S, D, H = 4, 2048, 1024, 2048
SBLK = 512
NSB = S // SBLK
EBLK = 256
NEB = S // EBLK
GP = 2056
NTILES = 32
CHUNK = S // (NTILES // B)
SUB = 16
NBUF = 4
DEPTH = 3


_HI = jax.lax.Precision.HIGHEST


def _stage1_body(x_ref, w1_ref, b1_ref, w2_ref, logits_ref, c_ref, carry_ref):
    x = x_ref[0]
    xb = x.astype(jnp.bfloat16)
    h = jax.lax.dot_general(xb, w1_ref[...], (((1,), (0,)), ((), ())),
                            preferred_element_type=jnp.float32)
    h = jnp.maximum(h + b1_ref[...], 0.0)
    hb = h.astype(jnp.bfloat16).astype(jnp.float32)
    w2f = w2_ref[...].astype(jnp.float32)
    logits = jnp.sum(hb * w2f, axis=1)
    logits_ref[...] = logits.reshape(1, 1, 1, SBLK)

    r = jax.lax.broadcasted_iota(jnp.int32, (SBLK, SBLK), 0)
    c = jax.lax.broadcasted_iota(jnp.int32, (SBLK, SBLK), 1)
    ltri = (r >= c).astype(jnp.bfloat16)
    cs = jax.lax.dot_general(ltri, xb, (((1,), (0,)), ((), ())),
                             preferred_element_type=jnp.float32)

    @pl.when(pl.program_id(1) == 0)
    def _():
        carry_ref[...] = jnp.zeros_like(carry_ref)

    total = cs + carry_ref[...]
    c_ref[0] = total
    carry_ref[...] = total[SBLK - 1:SBLK, :]


def _stage1(hidden, W1b, b1r, w2rb):
    return pl.pallas_call(
        _stage1_body,
        grid=(B, NSB),
        in_specs=[
            pl.BlockSpec((1, SBLK, D), lambda b, s: (b, s, 0)),
            pl.BlockSpec((D, H), lambda b, s: (0, 0)),
            pl.BlockSpec((1, H), lambda b, s: (0, 0)),
            pl.BlockSpec((1, H), lambda b, s: (0, 0)),
        ],
        out_specs=[
            pl.BlockSpec((1, 1, 1, SBLK), lambda b, s: (b, s, 0, 0)),
            pl.BlockSpec((1, SBLK, D), lambda b, s: (b, s, 0)),
        ],
        out_shape=[
            jax.ShapeDtypeStruct((B, NSB, 1, SBLK), jnp.float32),
            jax.ShapeDtypeStruct((B, S, D), jnp.float32),
        ],
        scratch_shapes=[pltpu.VMEM((1, D), jnp.float32)],
        compiler_params=pltpu.CompilerParams(
            dimension_semantics=("arbitrary", "arbitrary")),
    )(hidden, W1b, b1r, w2rb)


def _lane_cumsum(x):
    n = x.shape[1]
    sh = 1
    while sh < n:
        x = x + jnp.concatenate(
            [jnp.zeros(x.shape[:1] + (sh,), x.dtype), x[:, :-sh]], axis=1)
        sh *= 2
    return x


def _stage2_body(logits_ref, u_ref, b2_ref, g_ref, inv_ref, nb_ref):
    logits = logits_ref[...] + b2_ref[0, 0]
    probs = 1.0 / (1.0 + jnp.exp(-logits))
    u = u_ref[...]
    noisy = (jnp.log(probs + EPS) - jnp.log(1.0 - probs + EPS)
             + jnp.log(u) - jnp.log(1.0 - u)) / TEMP
    soft = 1.0 / (1.0 + jnp.exp(-noisy))
    hard = (soft > THRESHOLD)
    hardi = hard.astype(jnp.int32)
    nb_ref[0, 0] = jnp.sum(hardi.astype(jnp.float32))

    seg = _lane_cumsum(hardi) - hardi

    for b in range(B):
        segb = seg[b:b + 1, :]
        e_cols = []
        for k in range(NEB):
            nv = (jax.lax.broadcasted_iota(jnp.int32, (EBLK, 1), 0)
                  + k * EBLK)
            cmp = (segb <= nv).astype(jnp.int32)
            e_cols.append(jnp.sum(cmp, axis=1, keepdims=True))
        e_b = jnp.concatenate(e_cols, axis=0)
        g_ref[b] = (b * S - 1) + e_b
        e_prev = jnp.concatenate(
            [jnp.zeros((1, 1), jnp.int32), e_b[:-1]], axis=0)
        cnt = (e_b - e_prev).astype(jnp.float32)
        inv_ref[b] = 1.0 / (cnt + 1e-9)


def _stage2(logits, u, b2r):
    return pl.pallas_call(
        _stage2_body,
        in_specs=[
            pl.BlockSpec((B, S), lambda: (0, 0)),
            pl.BlockSpec((B, S), lambda: (0, 0)),
            pl.BlockSpec(memory_space=pltpu.SMEM),
        ],
        out_specs=[
            pl.BlockSpec((B, S, 1), lambda: (0, 0, 0)),
            pl.BlockSpec((B, S, 1), lambda: (0, 0, 0)),
            pl.BlockSpec(memory_space=pltpu.SMEM),
        ],
        out_shape=[
            jax.ShapeDtypeStruct((B, S, 1), jnp.int32),
            jax.ShapeDtypeStruct((B, S, 1), jnp.float32),
            jax.ShapeDtypeStruct((1, 1), jnp.float32),
        ],
    )(logits, u, b2r)


def _sc_body(cpad_hbm, g_hbm, out_hbm, idx_v, *bufs_and_sems):
    bufs = bufs_and_sems[:NBUF]
    gsems = bufs_and_sems[NBUF:2 * NBUF]
    wsems = bufs_and_sems[2 * NBUF:3 * NBUF]
    wid = lax.axis_index("s") * 2 + lax.axis_index("c")
    nk = CHUNK // SUB
    ibase = wid * CHUNK
    pltpu.sync_copy(g_hbm.at[pl.ds(wid * nk, nk)], idx_v)
    gpend = [None] * NBUF
    wpend = [None] * NBUF
    for k in range(min(DEPTH, nk)):
        gpend[k % NBUF] = pltpu.async_copy(
            cpad_hbm.at[idx_v.at[k]], bufs[k % NBUF], gsems[k % NBUF])
    for k in range(nk):
        i = k % NBUF
        gpend[i].wait()
        gpend[i] = None
        kk = k + DEPTH
        if kk < nk:
            j = kk % NBUF
            if wpend[j] is not None:
                wpend[j].wait()
                wpend[j] = None
            gpend[j] = pltpu.async_copy(
                cpad_hbm.at[idx_v.at[kk]], bufs[j], gsems[j])
        wpend[i] = pltpu.async_copy(
            bufs[i], out_hbm.at[pl.ds(ibase + k * SUB, SUB)], wsems[i])
    for w in wpend:
        if w is not None:
            w.wait()


def _sc_gather(cpad_flat, g_win):
    mesh = plsc.VectorSubcoreMesh(core_axis_name="c", subcore_axis_name="s")
    f = functools.partial(
        pl.kernel,
        mesh=mesh,
        out_type=jax.ShapeDtypeStruct((B * S, D), jnp.float32),
        scratch_types=(
            [pltpu.VMEM((CHUNK // SUB, SUB), jnp.int32)]
            + [pltpu.VMEM((SUB, D), jnp.float32)] * NBUF
            + [pltpu.SemaphoreType.DMA] * (2 * NBUF)
        ),
    )(_sc_body)
    return f(cpad_flat, g_win)


def _stage4_body(g_ref, seed_ref, inv_ref, out_ref):
    x = g_ref[0]
    prev = seed_ref[0, 7:8, :]
    prev = jnp.where(pl.program_id(1) == 0, 0.0, prev)
    shifted = jnp.concatenate([prev, x[:SBLK - 1, :]], axis=0)
    out_ref[0] = (x - shifted) * inv_ref[0]


def _stage4(gath, inv_col):
    return pl.pallas_call(
        _stage4_body,
        grid=(B, NSB),
        in_specs=[
            pl.BlockSpec((1, SBLK, D), lambda b, s: (b, s, 0)),
            pl.BlockSpec((1, 8, D),
                         lambda b, s: (b, jnp.maximum(s * (SBLK // 8) - 1, 0),
                                       0)),
            pl.BlockSpec((1, SBLK, 1), lambda b, s: (b, s, 0)),
        ],
        out_specs=pl.BlockSpec((1, SBLK, D), lambda b, s: (b, s, 0)),
        out_shape=jax.ShapeDtypeStruct((B, S, D), jnp.float32),
    )(gath, gath, inv_col)


def kernel(hidden, W1, b1, W2, b2):
    u = jax.random.uniform(jax.random.key(42), (B, S),
                           minval=1e-6, maxval=1.0 - 1e-6, dtype=jnp.float32)
    logits4, C = _stage1(hidden, W1.astype(jnp.bfloat16), b1.reshape(1, H),
                         W2.reshape(1, H).astype(jnp.bfloat16))
    G2, invbc, nb = _stage2(logits4.reshape(B, S), u, b2.reshape(1, 1))

    gwin = G2.reshape(B * (S // SUB), SUB)
    gath = _sc_gather(C.reshape(B * S, D), gwin).reshape(B, S, D)
    pooled = _stage4(gath, invbc)

    k = nb[0, 0]
    n = jnp.asarray(B * S, jnp.float32)
    log_prob = (jax.lax.lgamma(n + 1.0) - jax.lax.lgamma(k + 1.0)
                - jax.lax.lgamma(n - k + 1.0)
                + k * jnp.log(PRIOR) + (n - k) * jnp.log(1.0 - PRIOR))
    loss = -log_prob / n
    return pooled, loss


# speedup vs baseline: 1.5491x; 1.0012x over previous
import functools

import jax
import jax.numpy as jnp
from jax import lax
from jax.experimental import pallas as pl
from jax.experimental.pallas import tpu as pltpu
from jax.experimental.pallas import tpu_sc as plsc

TEMP = 1.0
THRESHOLD = 0.5
PRIOR = 0.2
EPS = 1e-8

B, S, D, H = 4, 2048, 1024, 2048
SBLK = 512
NSB = S // SBLK
EBLK = 256
NEB = S // EBLK
GP = 2056
NTILES = 32
CHUNK = S // (NTILES // B)
SUB = 16
NBUF = 4
DEPTH = 3


_HI = jax.lax.Precision.HIGHEST


def _stage1_body(x_ref, w1_ref, b1_ref, w2_ref, logits_ref, c_ref, carry_ref):
    x = x_ref[0]
    xb = x.astype(jnp.bfloat16)
    h = jax.lax.dot_general(xb, w1_ref[...], (((1,), (0,)), ((), ())),
                            preferred_element_type=jnp.float32)
    h = jnp.maximum(h + b1_ref[...], 0.0)
    hb = h.astype(jnp.bfloat16).astype(jnp.float32)
    w2f = w2_ref[...].astype(jnp.float32)
    logits = jnp.sum(hb * w2f, axis=1)
    logits_ref[...] = logits.reshape(1, 1, 1, SBLK)

    r = jax.lax.broadcasted_iota(jnp.int32, (SBLK, SBLK), 0)
    c = jax.lax.broadcasted_iota(jnp.int32, (SBLK, SBLK), 1)
    ltri = (r >= c).astype(jnp.bfloat16)
    cs = jax.lax.dot_general(ltri, xb, (((1,), (0,)), ((), ())),
                             preferred_element_type=jnp.float32)

    @pl.when(pl.program_id(1) == 0)
    def _():
        carry_ref[...] = jnp.zeros_like(carry_ref)

    total = cs + carry_ref[...]
    c_ref[0] = total
    carry_ref[...] = total[SBLK - 1:SBLK, :]


def _stage1(hidden, W1b, b1r, w2rb):
    return pl.pallas_call(
        _stage1_body,
        grid=(B, NSB),
        in_specs=[
            pl.BlockSpec((1, SBLK, D), lambda b, s: (b, s, 0)),
            pl.BlockSpec((D, H), lambda b, s: (0, 0)),
            pl.BlockSpec((1, H), lambda b, s: (0, 0)),
            pl.BlockSpec((1, H), lambda b, s: (0, 0)),
        ],
        out_specs=[
            pl.BlockSpec((1, 1, 1, SBLK), lambda b, s: (b, s, 0, 0)),
            pl.BlockSpec((1, SBLK, D), lambda b, s: (b, s, 0)),
        ],
        out_shape=[
            jax.ShapeDtypeStruct((B, NSB, 1, SBLK), jnp.float32),
            jax.ShapeDtypeStruct((B, S, D), jnp.float32),
        ],
        scratch_shapes=[pltpu.VMEM((1, D), jnp.float32)],
        compiler_params=pltpu.CompilerParams(
            dimension_semantics=("arbitrary", "arbitrary")),
    )(hidden, W1b, b1r, w2rb)


def _lane_cumsum(x):
    n = x.shape[1]
    sh = 1
    while sh < n:
        x = x + jnp.concatenate(
            [jnp.zeros(x.shape[:1] + (sh,), x.dtype), x[:, :-sh]], axis=1)
        sh *= 2
    return x


def _stage2_body(logits_ref, u_ref, b2_ref, g_ref, inv_ref, nb_ref):
    logits = logits_ref[...] + b2_ref[0, 0]
    probs = 1.0 / (1.0 + jnp.exp(-logits))
    u = u_ref[...]
    noisy = (jnp.log(probs + EPS) - jnp.log(1.0 - probs + EPS)
             + jnp.log(u) - jnp.log(1.0 - u)) / TEMP
    soft = 1.0 / (1.0 + jnp.exp(-noisy))
    hard = (soft > THRESHOLD)
    hardi = hard.astype(jnp.int32)
    nb_ref[0, 0] = jnp.sum(hardi.astype(jnp.float32))

    seg = _lane_cumsum(hardi) - hardi

    for b in range(B):
        segb = seg[b:b + 1, :]
        e_cols = []
        for k in range(NEB):
            nv = (jax.lax.broadcasted_iota(jnp.int32, (EBLK, 1), 0)
                  + k * EBLK)
            cmp = (segb <= nv).astype(jnp.int32)
            e_cols.append(jnp.sum(cmp, axis=1, keepdims=True))
        e_b = jnp.concatenate(e_cols, axis=0)
        g_ref[b] = (b * S - 1) + e_b
        e_prev = jnp.concatenate(
            [jnp.zeros((1, 1), jnp.int32), e_b[:-1]], axis=0)
        cnt = (e_b - e_prev).astype(jnp.float32)
        inv_ref[b] = 1.0 / (cnt + 1e-9)


def _stage2(logits, u, b2r):
    return pl.pallas_call(
        _stage2_body,
        in_specs=[
            pl.BlockSpec((B, S), lambda: (0, 0)),
            pl.BlockSpec((B, S), lambda: (0, 0)),
            pl.BlockSpec(memory_space=pltpu.SMEM),
        ],
        out_specs=[
            pl.BlockSpec((B, S, 1), lambda: (0, 0, 0)),
            pl.BlockSpec((B, S, 1), lambda: (0, 0, 0)),
            pl.BlockSpec(memory_space=pltpu.SMEM),
        ],
        out_shape=[
            jax.ShapeDtypeStruct((B, S, 1), jnp.int32),
            jax.ShapeDtypeStruct((B, S, 1), jnp.float32),
            jax.ShapeDtypeStruct((1, 1), jnp.float32),
        ],
    )(logits, u, b2r)


def _sc_body(cpad_hbm, g_hbm, out_hbm, idx_v, *bufs_and_sems):
    bufs = bufs_and_sems[:NBUF]
    gsems = bufs_and_sems[NBUF:2 * NBUF]
    wsems = bufs_and_sems[2 * NBUF:3 * NBUF]
    wid = lax.axis_index("s") * 2 + lax.axis_index("c")
    nk = CHUNK // SUB
    ibase = wid * CHUNK
    pltpu.sync_copy(g_hbm.at[pl.ds(wid * nk, nk)], idx_v)
    gpend = [None] * NBUF
    wpend = [None] * NBUF
    for k in range(min(DEPTH, nk)):
        gpend[k % NBUF] = pltpu.async_copy(
            cpad_hbm.at[idx_v.at[k]], bufs[k % NBUF], gsems[k % NBUF])
    for k in range(nk):
        i = k % NBUF
        gpend[i].wait()
        gpend[i] = None
        kk = k + DEPTH
        if kk < nk:
            j = kk % NBUF
            if wpend[j] is not None:
                wpend[j].wait()
                wpend[j] = None
            gpend[j] = pltpu.async_copy(
                cpad_hbm.at[idx_v.at[kk]], bufs[j], gsems[j])
        wpend[i] = pltpu.async_copy(
            bufs[i], out_hbm.at[pl.ds(ibase + k * SUB, SUB)], wsems[i])
    for w in wpend:
        if w is not None:
            w.wait()


def _sc_gather(cpad_flat, g_win):
    mesh = plsc.VectorSubcoreMesh(core_axis_name="c", subcore_axis_name="s")
    f = functools.partial(
        pl.kernel,
        mesh=mesh,
        out_type=jax.ShapeDtypeStruct((B * S, D), jnp.float32),
        scratch_types=(
            [pltpu.VMEM((CHUNK // SUB, SUB), jnp.int32)]
            + [pltpu.VMEM((SUB, D), jnp.float32)] * NBUF
            + [pltpu.SemaphoreType.DMA] * (2 * NBUF)
        ),
    )(_sc_body)
    return f(cpad_flat, g_win)


SBLK4 = 256
NSB4 = S // SBLK4


def _stage4_body(g_ref, seed_ref, inv_ref, out_ref):
    x = g_ref[0]
    prev = seed_ref[0, 7:8, :]
    prev = jnp.where(pl.program_id(1) == 0, 0.0, prev)
    shifted = jnp.concatenate([prev, x[:SBLK4 - 1, :]], axis=0)
    out_ref[0] = (x - shifted) * inv_ref[0]


def _stage4(gath, inv_col):
    return pl.pallas_call(
        _stage4_body,
        grid=(B, NSB4),
        in_specs=[
            pl.BlockSpec((1, SBLK4, D), lambda b, s: (b, s, 0)),
            pl.BlockSpec((1, 8, D),
                         lambda b, s: (b, jnp.maximum(s * (SBLK4 // 8) - 1, 0),
                                       0)),
            pl.BlockSpec((1, SBLK4, 1), lambda b, s: (b, s, 0)),
        ],
        out_specs=pl.BlockSpec((1, SBLK4, D), lambda b, s: (b, s, 0)),
        out_shape=jax.ShapeDtypeStruct((B, S, D), jnp.float32),
    )(gath, gath, inv_col)


def kernel(hidden, W1, b1, W2, b2):
    u = jax.random.uniform(jax.random.key(42), (B, S),
                           minval=1e-6, maxval=1.0 - 1e-6, dtype=jnp.float32)
    logits4, C = _stage1(hidden, W1.astype(jnp.bfloat16), b1.reshape(1, H),
                         W2.reshape(1, H).astype(jnp.bfloat16))
    G2, invbc, nb = _stage2(logits4.reshape(B, S), u, b2.reshape(1, 1))

    gwin = G2.reshape(B * (S // SUB), SUB)
    gath = _sc_gather(C.reshape(B * S, D), gwin).reshape(B, S, D)
    pooled = _stage4(gath, invbc)

    k = nb[0, 0]
    n = jnp.asarray(B * S, jnp.float32)
    log_prob = (jax.lax.lgamma(n + 1.0) - jax.lax.lgamma(k + 1.0)
                - jax.lax.lgamma(n - k + 1.0)
                + k * jnp.log(PRIOR) + (n - k) * jnp.log(1.0 - PRIOR))
    loss = -log_prob / n
    return pooled, loss


# speedup vs baseline: 1.5504x; 1.0009x over previous
import functools

import jax
import jax.numpy as jnp
from jax import lax
from jax.experimental import pallas as pl
from jax.experimental.pallas import tpu as pltpu
from jax.experimental.pallas import tpu_sc as plsc

TEMP = 1.0
THRESHOLD = 0.5
PRIOR = 0.2
EPS = 1e-8

B, S, D, H = 4, 2048, 1024, 2048
SBLK = 512
NSB = S // SBLK
EBLK = 256
NEB = S // EBLK
NTILES = 32
CHUNK = S // (NTILES // B)
SUB = 16
NBUF = 4
DEPTH = 3


def _stage1_body(x_ref, w1_ref, b1_ref, w2_ref, logits_ref, c_ref, carry_ref):
    x = x_ref[0]
    xb = x.astype(jnp.bfloat16)
    h = jax.lax.dot_general(xb, w1_ref[...], (((1,), (0,)), ((), ())),
                            preferred_element_type=jnp.float32)
    h = jnp.maximum(h + b1_ref[...], 0.0)
    hb = h.astype(jnp.bfloat16).astype(jnp.float32)
    w2f = w2_ref[...].astype(jnp.float32)
    logits = jnp.sum(hb * w2f, axis=1)
    logits_ref[...] = logits.reshape(1, 1, 1, SBLK)

    r = jax.lax.broadcasted_iota(jnp.int32, (SBLK, SBLK), 0)
    c = jax.lax.broadcasted_iota(jnp.int32, (SBLK, SBLK), 1)
    ltri = (r >= c).astype(jnp.bfloat16)
    cs = jax.lax.dot_general(ltri, xb, (((1,), (0,)), ((), ())),
                             preferred_element_type=jnp.float32)

    @pl.when(pl.program_id(1) == 0)
    def _():
        carry_ref[...] = jnp.zeros_like(carry_ref)

    total = cs + carry_ref[...]
    c_ref[0] = total
    carry_ref[...] = total[SBLK - 1:SBLK, :]


def _stage1(hidden, W1b, b1r, w2rb):
    return pl.pallas_call(
        _stage1_body,
        grid=(B, NSB),
        in_specs=[
            pl.BlockSpec((1, SBLK, D), lambda b, s: (b, s, 0)),
            pl.BlockSpec((D, H), lambda b, s: (0, 0)),
            pl.BlockSpec((1, H), lambda b, s: (0, 0)),
            pl.BlockSpec((1, H), lambda b, s: (0, 0)),
        ],
        out_specs=[
            pl.BlockSpec((1, 1, 1, SBLK), lambda b, s: (b, s, 0, 0)),
            pl.BlockSpec((1, SBLK, D), lambda b, s: (b, s, 0)),
        ],
        out_shape=[
            jax.ShapeDtypeStruct((B, NSB, 1, SBLK), jnp.float32),
            jax.ShapeDtypeStruct((B, S, D), jnp.float32),
        ],
        scratch_shapes=[pltpu.VMEM((1, D), jnp.float32)],
        compiler_params=pltpu.CompilerParams(
            dimension_semantics=("arbitrary", "arbitrary")),
    )(hidden, W1b, b1r, w2rb)


def _lane_cumsum(x):
    n = x.shape[1]
    sh = 1
    while sh < n:
        x = x + jnp.concatenate(
            [jnp.zeros(x.shape[:1] + (sh,), x.dtype), x[:, :-sh]], axis=1)
        sh *= 2
    return x


def _stage2_body(logits_ref, u_ref, b2_ref, g_ref, inv_ref, nb_ref):
    logits = logits_ref[...] + b2_ref[0, 0]
    probs = 1.0 / (1.0 + jnp.exp(-logits))
    u = u_ref[...]
    noisy = (jnp.log(probs + EPS) - jnp.log(1.0 - probs + EPS)
             + jnp.log(u) - jnp.log(1.0 - u)) / TEMP
    soft = 1.0 / (1.0 + jnp.exp(-noisy))
    hard = (soft > THRESHOLD)
    hardi = hard.astype(jnp.int32)
    nb_ref[0, 0] = jnp.sum(hardi.astype(jnp.float32))

    seg = _lane_cumsum(hardi) - hardi

    for b in range(B):
        segb = seg[b:b + 1, :]
        e_cols = []
        for k in range(NEB):
            nv = (jax.lax.broadcasted_iota(jnp.int32, (EBLK, 1), 0)
                  + k * EBLK)
            cmp = (segb <= nv).astype(jnp.int32)
            e_cols.append(jnp.sum(cmp, axis=1, keepdims=True))
        e_b = jnp.concatenate(e_cols, axis=0)
        g_ref[b] = (b * S - 1) + e_b
        e_prev = jnp.concatenate(
            [jnp.zeros((1, 1), jnp.int32), e_b[:-1]], axis=0)
        cnt = (e_b - e_prev).astype(jnp.float32)
        inv_ref[b] = 1.0 / (cnt + 1e-9)


def _stage2(logits, u, b2r):
    return pl.pallas_call(
        _stage2_body,
        in_specs=[
            pl.BlockSpec((B, S), lambda: (0, 0)),
            pl.BlockSpec((B, S), lambda: (0, 0)),
            pl.BlockSpec(memory_space=pltpu.SMEM),
        ],
        out_specs=[
            pl.BlockSpec((B, S, 1), lambda: (0, 0, 0)),
            pl.BlockSpec((B, S, 1), lambda: (0, 0, 0)),
            pl.BlockSpec(memory_space=pltpu.SMEM),
        ],
        out_shape=[
            jax.ShapeDtypeStruct((B, S, 1), jnp.int32),
            jax.ShapeDtypeStruct((B, S, 1), jnp.float32),
            jax.ShapeDtypeStruct((1, 1), jnp.float32),
        ],
    )(logits, u, b2r)


def _sc_body(cpad_hbm, g_hbm, out_hbm, idx_v, *bufs_and_sems):
    bufs = bufs_and_sems[:NBUF]
    gsems = bufs_and_sems[NBUF:2 * NBUF]
    wsems = bufs_and_sems[2 * NBUF:3 * NBUF]
    wid = lax.axis_index("s") * 2 + lax.axis_index("c")
    nk = CHUNK // SUB
    ibase = wid * CHUNK
    pltpu.sync_copy(g_hbm.at[pl.ds(wid * nk, nk)], idx_v)
    gpend = [None] * NBUF
    wpend = [None] * NBUF
    for k in range(min(DEPTH, nk)):
        gpend[k % NBUF] = pltpu.async_copy(
            cpad_hbm.at[idx_v.at[k]], bufs[k % NBUF], gsems[k % NBUF])
    for k in range(nk):
        i = k % NBUF
        gpend[i].wait()
        gpend[i] = None
        kk = k + DEPTH
        if kk < nk:
            j = kk % NBUF
            if wpend[j] is not None:
                wpend[j].wait()
                wpend[j] = None
            gpend[j] = pltpu.async_copy(
                cpad_hbm.at[idx_v.at[kk]], bufs[j], gsems[j])
        wpend[i] = pltpu.async_copy(
            bufs[i], out_hbm.at[pl.ds(ibase + k * SUB, SUB)], wsems[i])
    for w in wpend:
        if w is not None:
            w.wait()


def _sc_gather(cpad_flat, g_win):
    mesh = plsc.VectorSubcoreMesh(core_axis_name="c", subcore_axis_name="s")
    f = functools.partial(
        pl.kernel,
        mesh=mesh,
        out_type=jax.ShapeDtypeStruct((B * S, D), jnp.float32),
        scratch_types=(
            [pltpu.VMEM((CHUNK // SUB, SUB), jnp.int32)]
            + [pltpu.VMEM((SUB, D), jnp.float32)] * NBUF
            + [pltpu.SemaphoreType.DMA] * (2 * NBUF)
        ),
    )(_sc_body)
    return f(cpad_flat, g_win)


SBLK4 = 256
NSB4 = S // SBLK4


def _stage4_body(g_ref, seed_ref, inv_ref, out_ref):
    x = g_ref[0]
    prev = seed_ref[0, 7:8, :]
    prev = jnp.where(pl.program_id(1) == 0, 0.0, prev)
    shifted = jnp.concatenate([prev, x[:SBLK4 - 1, :]], axis=0)
    out_ref[0] = (x - shifted) * inv_ref[0]


def _stage4(gath, inv_col):
    return pl.pallas_call(
        _stage4_body,
        grid=(B, NSB4),
        in_specs=[
            pl.BlockSpec((1, SBLK4, D), lambda b, s: (b, s, 0)),
            pl.BlockSpec((1, 8, D),
                         lambda b, s: (b, jnp.maximum(s * (SBLK4 // 8) - 1, 0),
                                       0)),
            pl.BlockSpec((1, SBLK4, 1), lambda b, s: (b, s, 0)),
        ],
        out_specs=pl.BlockSpec((1, SBLK4, D), lambda b, s: (b, s, 0)),
        out_shape=jax.ShapeDtypeStruct((B, S, D), jnp.float32),
    )(gath, gath, inv_col)


def kernel(hidden, W1, b1, W2, b2):
    u = jax.random.uniform(jax.random.key(42), (B, S),
                           minval=1e-6, maxval=1.0 - 1e-6, dtype=jnp.float32)
    logits4, C = _stage1(hidden, W1.astype(jnp.bfloat16), b1.reshape(1, H),
                         W2.reshape(1, H).astype(jnp.bfloat16))
    G2, invbc, nb = _stage2(logits4.reshape(B, S), u, b2.reshape(1, 1))

    gwin = G2.reshape(B * (S // SUB), SUB)
    gath = _sc_gather(C.reshape(B * S, D), gwin).reshape(B, S, D)
    pooled = _stage4(gath, invbc)

    k = nb[0, 0]
    n = jnp.asarray(B * S, jnp.float32)
    log_prob = (jax.lax.lgamma(n + 1.0) - jax.lax.lgamma(k + 1.0)
                - jax.lax.lgamma(n - k + 1.0)
                + k * jnp.log(PRIOR) + (n - k) * jnp.log(1.0 - PRIOR))
    loss = -log_prob / n
    return pooled, loss
